# baseline (speedup 1.0000x reference)
import functools

import jax
import jax.numpy as jnp
from jax import lax
from jax.experimental import pallas as pl
from jax.experimental.pallas import tpu as pltpu
from jax.experimental.pallas import tpu_sc as plsc

NC = 2
NS = 16
CH = 128
NB = 3
QR = 6
UN = 6


def _sc_partials(x, eidx):
    N, D = x.shape
    nchk = eidx.shape[1] // CH
    NW = NC * NS
    NL = nchk // NW
    extra = nchk - NL * NW
    assert extra < NW and NL > QR
    rpt = (N // NS) // 8 * 8
    rem = N - NS * rpt
    assert rem % 8 == 0 and rem >= 0

    mesh = plsc.VectorSubcoreMesh(core_axis_name="c", subcore_axis_name="s")

    @functools.partial(
        pl.kernel,
        out_type=jax.ShapeDtypeStruct((NC * N, D), jnp.float32),
        mesh=mesh,
        scratch_types=[
            [pltpu.VMEM((2, CH), jnp.int32) for _ in range(QR)],
            pltpu.VMEM((NB * CH, D), jnp.float32),
            pltpu.VMEM_SHARED((N, D), jnp.float32),
            [pltpu.SemaphoreType.DMA for _ in range(NB)],
            [pltpu.SemaphoreType.DMA for _ in range(QR)],
            pltpu.SemaphoreType.DMA,
        ],
    )
    def sc_kernel(x_hbm, eidx_hbm, out_hbm,
                  sidx, rows_all, acc, gsems, isems, nsem):
        rows = [rows_all.at[pl.ds(b * CH, CH)] for b in range(NB)]
        cid = lax.axis_index("c")
        sid = lax.axis_index("s")
        wid = sid * NC + cid
        cbase = wid * NL
        rbase = sid * rpt

        pltpu.async_copy(x_hbm.at[pl.ds(rbase, rpt)],
                         acc.at[pl.ds(rbase, rpt)], nsem)
        if rem:
            @pl.when(sid == NS - 1)
            def _():
                pltpu.async_copy(x_hbm.at[pl.ds(NS * rpt, rem)],
                                 acc.at[pl.ds(NS * rpt, rem)], nsem)

        def drain(sem, buf_ref):
            pltpu.make_async_copy(x_hbm.at[pl.ds(0, CH)], buf_ref, sem).wait()

        def drain_idx(q):
            pltpu.make_async_copy(eidx_hbm.at[pl.ds(0, 2), pl.ds(0, CH)],
                                  sidx[q], isems[q]).wait()

        def fire_idx(j, q):
            pltpu.async_copy(
                eidx_hbm.at[pl.ds(0, 2), pl.ds((cbase + j) * CH, CH)],
                sidx[q], isems[q])

        def fire_gather(b, q):
            pltpu.async_copy(x_hbm.at[sidx[q].at[0]], rows[b], gsems[b])

        for q in range(QR):
            fire_idx(q, q)
        for b in range(NB):
            drain_idx(b)
            fire_gather(b, b)

        pltpu.make_async_copy(x_hbm.at[pl.ds(0, rpt)],
                              acc.at[pl.ds(0, rpt)], nsem).wait()
        if rem:
            @pl.when(sid == NS - 1)
            def _():
                pltpu.make_async_copy(x_hbm.at[pl.ds(0, rem)],
                                      acc.at[pl.ds(0, rem)], nsem).wait()
        plsc.subcore_barrier()

        def step(i, b, q, do_fidx, do_fg):
            drain(gsems[b], rows[b])
            pltpu.sync_copy(rows[b], acc.at[sidx[q].at[1]], add=True)
            if do_fidx:
                fire_idx(i + QR, q)
            if do_fg:
                drain_idx((q + NB) % QR)
                fire_gather(b, (q + NB) % QR)

        n_grp = (NL - QR) // UN
        n_peeled = n_grp * UN

        def outer(g, carry):
            for u in range(UN):
                step(g * UN + u, u % NB, u % QR, True, True)
            return carry

        lax.fori_loop(0, n_grp, outer, 0)

        for i in range(n_peeled, NL):
            step(i, i % NB, i % QR, i + QR < NL, i + NB < NL)

        if extra:
            @pl.when(wid < extra)
            def _():
                jj = nchk - extra + wid
                pltpu.sync_copy(
                    eidx_hbm.at[pl.ds(0, 2), pl.ds(jj * CH, CH)], sidx[0])
                pltpu.async_copy(x_hbm.at[sidx[0].at[0]], rows[0],
                                 gsems[0]).wait()
                pltpu.sync_copy(rows[0], acc.at[sidx[0].at[1]], add=True)

        plsc.subcore_barrier()

        pltpu.sync_copy(acc.at[pl.ds(rbase, rpt)],
                        out_hbm.at[pl.ds(cid * N + rbase, rpt)])
        if rem:
            @pl.when(sid == NS - 1)
            def _():
                pltpu.sync_copy(acc.at[pl.ds(NS * rpt, rem)],
                                out_hbm.at[pl.ds(cid * N + NS * rpt, rem)])

    return sc_kernel(x, eidx)


def _combine(partials, x):
    N, D = x.shape
    br = 1000
    grid = N // br
    assert br * grid == N

    def body(p0_ref, p1_ref, x_ref, o_ref):
        o_ref[...] = p0_ref[...] + p1_ref[...] - x_ref[...]

    return pl.pallas_call(
        body,
        grid=(grid,),
        in_specs=[
            pl.BlockSpec((br, D), lambda i: (i, 0)),
            pl.BlockSpec((br, D), lambda i: (i + grid, 0)),
            pl.BlockSpec((br, D), lambda i: (i, 0)),
        ],
        out_specs=pl.BlockSpec((br, D), lambda i: (i, 0)),
        out_shape=jax.ShapeDtypeStruct((N, D), jnp.float32),
    )(partials, partials, x)


def kernel(x, edge_index):
    E = edge_index.shape[1]
    assert E % CH == 0
    partials = _sc_partials(x, edge_index)
    return _combine(partials, x)


# speedup vs baseline: 1.0214x; 1.0214x over previous
import functools

import jax
import jax.numpy as jnp
from jax import lax
from jax.experimental import pallas as pl
from jax.experimental.pallas import tpu as pltpu
from jax.experimental.pallas import tpu_sc as plsc

NC = 2
NS = 16
CH = 128
NB = 3
QR = 6
UN = 6


def _sc_partials(x, eidx):
    N, D = x.shape
    nchk = eidx.shape[1] // CH
    NW = NC * NS
    NL = nchk // NW
    extra = nchk - NL * NW
    assert extra < NW and NL > QR
    rpt = (N // NS) // 8 * 8
    rem = N - NS * rpt
    assert rem % 8 == 0 and rem >= 0

    mesh = plsc.VectorSubcoreMesh(core_axis_name="c", subcore_axis_name="s")

    @functools.partial(
        pl.kernel,
        out_type=jax.ShapeDtypeStruct((NC * N, D), jnp.float32),
        mesh=mesh,
        scratch_types=[
            [pltpu.VMEM((2, CH), jnp.int32) for _ in range(QR)],
            pltpu.VMEM((NB * CH, D), jnp.float32),
            pltpu.VMEM_SHARED((N, D), jnp.float32),
            [pltpu.SemaphoreType.DMA for _ in range(NB)],
            [pltpu.SemaphoreType.DMA for _ in range(QR)],
            pltpu.SemaphoreType.DMA,
        ],
    )
    def sc_kernel(x_hbm, eidx_hbm, out_hbm,
                  sidx, rows_all, acc, gsems, isems, nsem):
        rows = [rows_all.at[pl.ds(b * CH, CH)] for b in range(NB)]
        cid = lax.axis_index("c")
        sid = lax.axis_index("s")
        wid = sid * NC + cid
        cbase = wid * NL
        rbase = sid * rpt

        pltpu.async_copy(x_hbm.at[pl.ds(rbase, rpt)],
                         acc.at[pl.ds(rbase, rpt)], nsem)
        if rem:
            @pl.when(sid == NS - 1)
            def _():
                pltpu.async_copy(x_hbm.at[pl.ds(NS * rpt, rem)],
                                 acc.at[pl.ds(NS * rpt, rem)], nsem)

        def drain(sem, buf_ref):
            pltpu.make_async_copy(x_hbm.at[pl.ds(0, CH)], buf_ref, sem).wait()

        def drain_idx(q):
            pltpu.make_async_copy(eidx_hbm.at[pl.ds(0, 2), pl.ds(0, CH)],
                                  sidx[q], isems[q]).wait()

        def fire_idx(j, q):
            pltpu.async_copy(
                eidx_hbm.at[pl.ds(0, 2), pl.ds((cbase + j) * CH, CH)],
                sidx[q], isems[q])

        def fire_gather(b, q):
            pltpu.async_copy(x_hbm.at[sidx[q].at[0]], rows[b], gsems[b])

        for q in range(QR):
            fire_idx(q, q)
        for b in range(NB):
            drain_idx(b)
            fire_gather(b, b)

        pltpu.make_async_copy(x_hbm.at[pl.ds(0, rpt)],
                              acc.at[pl.ds(0, rpt)], nsem).wait()
        if rem:
            @pl.when(sid == NS - 1)
            def _():
                pltpu.make_async_copy(x_hbm.at[pl.ds(0, rem)],
                                      acc.at[pl.ds(0, rem)], nsem).wait()
        plsc.subcore_barrier()

        def step(i, b, q, do_fidx, do_fg):
            drain(gsems[b], rows[b])
            pltpu.sync_copy(rows[b], acc.at[sidx[q].at[1]], add=True)
            if do_fidx:
                fire_idx(i + QR, q)
            if do_fg:
                drain_idx((q + NB) % QR)
                fire_gather(b, (q + NB) % QR)

        n_grp = (NL - QR) // UN
        n_peeled = n_grp * UN

        def outer(g, carry):
            for u in range(UN):
                step(g * UN + u, u % NB, u % QR, True, True)
            return carry

        lax.fori_loop(0, n_grp, outer, 0)

        for i in range(n_peeled, NL):
            step(i, i % NB, i % QR, i + QR < NL, i + NB < NL)

        if extra:
            @pl.when(wid < extra)
            def _():
                jj = nchk - extra + wid
                pltpu.sync_copy(
                    eidx_hbm.at[pl.ds(0, 2), pl.ds(jj * CH, CH)], sidx[0])
                pltpu.async_copy(x_hbm.at[sidx[0].at[0]], rows[0],
                                 gsems[0]).wait()
                pltpu.sync_copy(rows[0], acc.at[sidx[0].at[1]], add=True)

        plsc.subcore_barrier()

        pltpu.sync_copy(acc.at[pl.ds(rbase, rpt)],
                        out_hbm.at[pl.ds(cid * N + rbase, rpt)])
        if rem:
            @pl.when(sid == NS - 1)
            def _():
                pltpu.sync_copy(acc.at[pl.ds(NS * rpt, rem)],
                                out_hbm.at[pl.ds(cid * N + NS * rpt, rem)])

    return sc_kernel(x, eidx)


def _combine(partials, x):
    N, D = x.shape
    br = 5000
    grid = N // br
    assert br * grid == N

    def body(p0_ref, p1_ref, x_ref, o_ref):
        o_ref[...] = p0_ref[...] + p1_ref[...] - x_ref[...]

    return pl.pallas_call(
        body,
        grid=(grid,),
        in_specs=[
            pl.BlockSpec((br, D), lambda i: (i, 0)),
            pl.BlockSpec((br, D), lambda i: (i + grid, 0)),
            pl.BlockSpec((br, D), lambda i: (i, 0)),
        ],
        out_specs=pl.BlockSpec((br, D), lambda i: (i, 0)),
        out_shape=jax.ShapeDtypeStruct((N, D), jnp.float32),
    )(partials, partials, x)


def kernel(x, edge_index):
    E = edge_index.shape[1]
    assert E % CH == 0
    partials = _sc_partials(x, edge_index)
    return _combine(partials, x)


# speedup vs baseline: 1.0230x; 1.0015x over previous
import functools

import jax
import jax.numpy as jnp
from jax import lax
from jax.experimental import pallas as pl
from jax.experimental.pallas import tpu as pltpu
from jax.experimental.pallas import tpu_sc as plsc

NC = 2
NS = 16
CH = 128
NB = 3
QR = 6
UN = 6


def _sc_partials(x, eidx):
    N, D = x.shape
    nchk = eidx.shape[1] // CH
    NW = NC * NS
    NL = nchk // NW
    extra = nchk - NL * NW
    assert extra < NW and NL > QR
    rpt = (N // NS) // 8 * 8
    rem = N - NS * rpt
    assert rem % 8 == 0 and rem >= 0

    mesh = plsc.VectorSubcoreMesh(core_axis_name="c", subcore_axis_name="s")

    @functools.partial(
        pl.kernel,
        out_type=jax.ShapeDtypeStruct((NC * N, D), jnp.float32),
        mesh=mesh,
        scratch_types=[
            [pltpu.VMEM((2, CH), jnp.int32) for _ in range(QR)],
            pltpu.VMEM((NB * CH, D), jnp.float32),
            pltpu.VMEM_SHARED((N, D), jnp.float32),
            [pltpu.SemaphoreType.DMA for _ in range(NB)],
            [pltpu.SemaphoreType.DMA for _ in range(QR)],
            [pltpu.SemaphoreType.DMA for _ in range(NB)],
            pltpu.SemaphoreType.DMA,
        ],
    )
    def sc_kernel(x_hbm, eidx_hbm, out_hbm,
                  sidx, rows_all, acc, gsems, isems, ssems, nsem):
        rows = [rows_all.at[pl.ds(b * CH, CH)] for b in range(NB)]
        cid = lax.axis_index("c")
        sid = lax.axis_index("s")
        wid = sid * NC + cid
        cbase = wid * NL
        rbase = sid * rpt

        pltpu.async_copy(x_hbm.at[pl.ds(rbase, rpt)],
                         acc.at[pl.ds(rbase, rpt)], nsem)
        if rem:
            @pl.when(sid == NS - 1)
            def _():
                pltpu.async_copy(x_hbm.at[pl.ds(NS * rpt, rem)],
                                 acc.at[pl.ds(NS * rpt, rem)], nsem)

        def drain(sem, buf_ref):
            pltpu.make_async_copy(x_hbm.at[pl.ds(0, CH)], buf_ref, sem).wait()

        def drain_idx(q):
            pltpu.make_async_copy(eidx_hbm.at[pl.ds(0, 2), pl.ds(0, CH)],
                                  sidx[q], isems[q]).wait()

        def fire_idx(j, q):
            pltpu.async_copy(
                eidx_hbm.at[pl.ds(0, 2), pl.ds((cbase + j) * CH, CH)],
                sidx[q], isems[q])

        def fire_gather(b, q):
            pltpu.async_copy(x_hbm.at[sidx[q].at[0]], rows[b], gsems[b])

        for q in range(QR):
            fire_idx(q, q)
        for b in range(NB):
            drain_idx(b)
            fire_gather(b, b)

        pltpu.make_async_copy(x_hbm.at[pl.ds(0, rpt)],
                              acc.at[pl.ds(0, rpt)], nsem).wait()
        if rem:
            @pl.when(sid == NS - 1)
            def _():
                pltpu.make_async_copy(x_hbm.at[pl.ds(0, rem)],
                                      acc.at[pl.ds(0, rem)], nsem).wait()
        plsc.subcore_barrier()

        def step(i, b, q, do_fidx, do_fg):
            drain(gsems[b], rows[b])
            pltpu.async_copy(rows[b], acc.at[sidx[q].at[1]], ssems[b],
                             add=True)
            if do_fidx:
                fire_idx(i + QR, q)
            if do_fg:
                drain_idx((q + NB) % QR)
                drain(ssems[b], rows[b])
                fire_gather(b, (q + NB) % QR)
            else:
                drain(ssems[b], rows[b])

        n_grp = (NL - QR) // UN
        n_peeled = n_grp * UN

        def outer(g, carry):
            for u in range(UN):
                step(g * UN + u, u % NB, u % QR, True, True)
            return carry

        lax.fori_loop(0, n_grp, outer, 0)

        for i in range(n_peeled, NL):
            step(i, i % NB, i % QR, i + QR < NL, i + NB < NL)

        if extra:
            @pl.when(wid < extra)
            def _():
                jj = nchk - extra + wid
                pltpu.sync_copy(
                    eidx_hbm.at[pl.ds(0, 2), pl.ds(jj * CH, CH)], sidx[0])
                pltpu.async_copy(x_hbm.at[sidx[0].at[0]], rows[0],
                                 gsems[0]).wait()
                pltpu.sync_copy(rows[0], acc.at[sidx[0].at[1]], add=True)

        plsc.subcore_barrier()

        pltpu.sync_copy(acc.at[pl.ds(rbase, rpt)],
                        out_hbm.at[pl.ds(cid * N + rbase, rpt)])
        if rem:
            @pl.when(sid == NS - 1)
            def _():
                pltpu.sync_copy(acc.at[pl.ds(NS * rpt, rem)],
                                out_hbm.at[pl.ds(cid * N + NS * rpt, rem)])

    return sc_kernel(x, eidx)


def _combine(partials, x):
    N, D = x.shape
    br = 5000
    grid = N // br
    assert br * grid == N

    def body(p0_ref, p1_ref, x_ref, o_ref):
        o_ref[...] = p0_ref[...] + p1_ref[...] - x_ref[...]

    return pl.pallas_call(
        body,
        grid=(grid,),
        in_specs=[
            pl.BlockSpec((br, D), lambda i: (i, 0)),
            pl.BlockSpec((br, D), lambda i: (i + grid, 0)),
            pl.BlockSpec((br, D), lambda i: (i, 0)),
        ],
        out_specs=pl.BlockSpec((br, D), lambda i: (i, 0)),
        out_shape=jax.ShapeDtypeStruct((N, D), jnp.float32),
    )(partials, partials, x)


def kernel(x, edge_index):
    E = edge_index.shape[1]
    assert E % CH == 0
    partials = _sc_partials(x, edge_index)
    return _combine(partials, x)
